# baseline (speedup 1.0000x reference)
import functools

import jax
import jax.numpy as jnp
from jax import lax
from jax.experimental import pallas as pl
from jax.experimental.pallas import tpu as pltpu
from jax.experimental.pallas import tpu_sc as plsc

N_SRC = 10000
N_TGT = 10000
E = 320000
D = 128
H = 128
OUT = 64
HW = 64

NPAD = 10240
DUMMY = 10000
STRIPE = NPAD // 16
CH1 = 158
CH3 = 79
EPAD = 16 * CH1 * 128
NBUF = 2

_f32 = jnp.float32
_i32 = jnp.int32


@functools.cache
def _mesh():
    return plsc.VectorSubcoreMesh(core_axis_name="c", subcore_axis_name="s")


def _zero_vec_loop(ref, n16):
    def body(i, _):
        ref[pl.ds(i * 16, 16)] = jnp.zeros((16,), _f32)
        return 0
    lax.fori_loop(0, n16, body, 0)


def _zero_rows(rows_v):
    n = 128 * HW // 16

    def body(i, _):
        r = i // (HW // 16)
        k = i % (HW // 16)
        rows_v[r, pl.ds(k * 16, 16)] = jnp.zeros((16,), _f32)
        return 0
    lax.fori_loop(0, n, body, 0)


def _fill_ones(ones_v):
    def body(i, _):
        ones_v[pl.ds(i * 16, 16)] = jnp.ones((16,), _f32)
        return 0
    lax.fori_loop(0, 8, body, 0)


def _accumulate(x_hbm, edges, tile, nch, si_v, di_v, bufs,
                ones_v, acc_sh, cnt_sh, gsems):
    pltpu.sync_copy(edges.at[0, tile], si_v)
    pltpu.sync_copy(edges.at[1, tile], di_v)

    def gsrc(j):
        return x_hbm.at[si_v.at[j]]

    def gstart(j, b):
        pltpu.async_copy(gsrc(j), bufs[b], gsems[b])

    def gwait(j, b):
        pltpu.make_async_copy(gsrc(j), bufs[b], gsems[b]).wait()

    def scatter(j, b):
        pltpu.sync_copy(bufs[b], acc_sh.at[di_v.at[j]], add=True)
        if cnt_sh is not None:
            pltpu.sync_copy(ones_v, cnt_sh.at[di_v.at[j]], add=True)

    ngroups = nch // NBUF
    rem = nch - NBUF * ngroups

    for b in range(NBUF):
        gstart(b, b)

    def body(g, _):
        for b in range(NBUF):
            j = NBUF * g + b
            gwait(j, b)
            scatter(j, b)
            jn = j + NBUF

            @pl.when(jn < nch)
            def _():
                gstart(jn, b)
        return 0
    lax.fori_loop(0, ngroups, body, 0)

    for b in range(rem):
        j = NBUF * ngroups + b
        gwait(j, b)
        scatter(j, b)


def _zero_acc_stripe(rows_v, acc_sh, t):
    for b in range(STRIPE // 128):
        pltpu.sync_copy(rows_v, acc_sh.at[pl.ds(t * STRIPE + b * 128, 128)])


def _p1_body_counts(xs_hbm, xt_hbm, st_e, ts_e,
                    sum_st, cnt_st, sum_ts, cnt_ts,
                    si_v, di_v, b0, b1, ones_v, zc_v, acc_sh, cnt_sh,
                    g0, g1):
    c = lax.axis_index("c")
    t = lax.axis_index("s")
    bufs = (b0, b1)
    gsems = (g0, g1)

    _zero_rows(b0)
    _fill_ones(ones_v)
    _zero_vec_loop(zc_v, STRIPE // 16)
    _zero_acc_stripe(b0, acc_sh, t)
    pltpu.sync_copy(zc_v, cnt_sh.at[pl.ds(t * STRIPE, STRIPE)])
    plsc.subcore_barrier()

    @pl.when(c == 0)
    def _():
        _accumulate(xs_hbm, st_e, t, CH1, si_v, di_v, bufs,
                    ones_v, acc_sh, cnt_sh, gsems)

    @pl.when(c == 1)
    def _():
        _accumulate(xt_hbm, ts_e, t, CH1, si_v, di_v, bufs,
                    ones_v, acc_sh, cnt_sh, gsems)

    plsc.subcore_barrier()
    sl = pl.ds(t * STRIPE, STRIPE)

    @pl.when(c == 0)
    def _():
        pltpu.sync_copy(acc_sh.at[sl], sum_st.at[sl])
        pltpu.sync_copy(cnt_sh.at[sl], cnt_st.at[sl])

    @pl.when(c == 1)
    def _():
        pltpu.sync_copy(acc_sh.at[sl], sum_ts.at[sl])
        pltpu.sync_copy(cnt_sh.at[sl], cnt_ts.at[sl])


def _p1_body_nocounts(xs_hbm, xt_hbm, st_e, ts_e,
                      sum_st, sum_ts,
                      si_v, di_v, b0, b1, acc_sh,
                      g0, g1):
    c = lax.axis_index("c")
    t = lax.axis_index("s")
    bufs = (b0, b1)
    gsems = (g0, g1)

    _zero_rows(b0)
    _zero_acc_stripe(b0, acc_sh, t)
    plsc.subcore_barrier()

    @pl.when(c == 0)
    def _():
        _accumulate(xs_hbm, st_e, t, CH1, si_v, di_v, bufs,
                    None, acc_sh, None, gsems)

    @pl.when(c == 1)
    def _():
        _accumulate(xt_hbm, ts_e, t, CH1, si_v, di_v, bufs,
                    None, acc_sh, None, gsems)

    plsc.subcore_barrier()
    sl = pl.ds(t * STRIPE, STRIPE)

    @pl.when(c == 0)
    def _():
        pltpu.sync_copy(acc_sh.at[sl], sum_st.at[sl])

    @pl.when(c == 1)
    def _():
        pltpu.sync_copy(acc_sh.at[sl], sum_ts.at[sl])


def _p3_body(xt0_hbm, ts_e, out,
             si_v, di_v, b0, b1, acc_sh,
             g0, g1):
    c = lax.axis_index("c")
    t = lax.axis_index("s")
    w = c * 16 + t
    bufs = (b0, b1)
    gsems = (g0, g1)

    _zero_rows(b0)
    _zero_acc_stripe(b0, acc_sh, t)
    plsc.subcore_barrier()

    _accumulate(xt0_hbm, ts_e, w, CH3, si_v, di_v, bufs,
                None, acc_sh, None, gsems)

    plsc.subcore_barrier()
    pltpu.sync_copy(acc_sh.at[pl.ds(t * STRIPE, STRIPE)],
                    out.at[c, pl.ds(t * STRIPE, STRIPE)])


def _ring_scratch(nch):
    return ([pltpu.VMEM((nch, 128), _i32),
             pltpu.VMEM((nch, 128), _i32)]
            + [pltpu.VMEM((128, HW), _f32) for _ in range(NBUF)])


_SEMS = [pltpu.SemaphoreType.DMA] * NBUF


@functools.cache
def _sc_pass1_counts():
    return pl.kernel(
        _p1_body_counts,
        out_type=(
            jax.ShapeDtypeStruct((NPAD, HW), _f32),
            jax.ShapeDtypeStruct((NPAD,), _f32),
            jax.ShapeDtypeStruct((NPAD, HW), _f32),
            jax.ShapeDtypeStruct((NPAD,), _f32),
        ),
        mesh=_mesh(),
        compiler_params=pltpu.CompilerParams(use_tc_tiling_on_sc=False),
        scratch_types=_ring_scratch(CH1) + [
            pltpu.VMEM((128,), _f32),
            pltpu.VMEM((STRIPE,), _f32),
            pltpu.VMEM_SHARED((NPAD, HW), _f32),
            pltpu.VMEM_SHARED((NPAD,), _f32),
        ] + _SEMS,
    )


@functools.cache
def _sc_pass1_nocounts():
    return pl.kernel(
        _p1_body_nocounts,
        out_type=(
            jax.ShapeDtypeStruct((NPAD, HW), _f32),
            jax.ShapeDtypeStruct((NPAD, HW), _f32),
        ),
        mesh=_mesh(),
        compiler_params=pltpu.CompilerParams(use_tc_tiling_on_sc=False),
        scratch_types=_ring_scratch(CH1) + [
            pltpu.VMEM_SHARED((NPAD, HW), _f32),
        ] + _SEMS,
    )


@functools.cache
def _sc_pass3():
    return pl.kernel(
        _p3_body,
        out_type=jax.ShapeDtypeStruct((2, NPAD, HW), _f32),
        mesh=_mesh(),
        compiler_params=pltpu.CompilerParams(use_tc_tiling_on_sc=False),
        scratch_types=_ring_scratch(CH3) + [
            pltpu.VMEM_SHARED((NPAD, HW), _f32),
        ] + _SEMS,
    )


def _lrelu(x):
    return jnp.where(x >= 0, x, 0.01 * x)


_BLK = 2048


def _tc_layer0_body(sum_ta, sum_tb, cnt_t, x_t, wl_t, bl_t, wr_t,
                    sum_sa, sum_sb, cnt_s, x_s, wl_s, bl_s, wr_s,
                    xt0a, xt0b, xs0):
    rt = 1.0 / jnp.maximum(cnt_t[...], 1.0)
    mean_t = jnp.concatenate([sum_ta[...], sum_tb[...]], axis=1) * rt[:, None]
    h_t = (jnp.dot(mean_t, wl_t[...].T, preferred_element_type=_f32)
           + bl_t[...][None, :]
           + jnp.dot(x_t[...], wr_t[...].T, preferred_element_type=_f32))
    a_t = _lrelu(h_t)
    xt0a[...] = a_t[:, :HW]
    xt0b[...] = a_t[:, HW:]
    rs = 1.0 / jnp.maximum(cnt_s[...], 1.0)
    mean_s = jnp.concatenate([sum_sa[...], sum_sb[...]], axis=1) * rs[:, None]
    h_s = (jnp.dot(mean_s, wl_s[...].T, preferred_element_type=_f32)
           + bl_s[...][None, :]
           + jnp.dot(x_s[...], wr_s[...].T, preferred_element_type=_f32))
    xs0[...] = _lrelu(h_s)


def _tc_layer1_body(pa0, pa1, pb0, pb1, cnt, xs0, wl, bl, wr, wo, bo, out):
    r = 1.0 / jnp.maximum(cnt[...], 1.0)
    mean = jnp.concatenate([pa0[...] + pa1[...], pb0[...] + pb1[...]],
                           axis=1) * r[:, None]
    h = (jnp.dot(mean, wl[...].T, preferred_element_type=_f32)
         + bl[...][None, :]
         + jnp.dot(xs0[...], wr[...].T, preferred_element_type=_f32))
    xs1 = _lrelu(h)
    out[...] = (jnp.dot(xs1, wo[...].T, preferred_element_type=_f32)
                + bo[...][None, :])


def _row_blk(shape_minor):
    return pl.BlockSpec((_BLK,) + shape_minor,
                        lambda i: (i,) + (0,) * len(shape_minor))


def _full(shape):
    return pl.BlockSpec(shape, lambda i: (0,) * len(shape))


def _prep_edges(ei):
    base = jnp.concatenate(
        [jnp.zeros((1, EPAD), _i32), jnp.full((1, EPAD), DUMMY, _i32)], axis=0)
    return base.at[:, :E].set(ei.astype(_i32))


def kernel(x_source, x_target, edge_index_st, edge_index_ts,
           l0_st_Wl, l0_st_bl, l0_st_Wr, l0_ts_Wl, l0_ts_bl, l0_ts_Wr,
           l1_st_Wl, l1_st_bl, l1_st_Wr, l1_ts_Wl, l1_ts_bl, l1_ts_Wr,
           W_out, b_out):
    st_e = _prep_edges(edge_index_st).reshape(2, 16, CH1, 128)
    ts_ef = _prep_edges(edge_index_ts)
    ts_e = ts_ef.reshape(2, 16, CH1, 128)
    ts3_e = ts_ef.reshape(2, 32, CH3, 128)

    xs_a, xs_b = x_source[:, :HW], x_source[:, HW:]
    xt_a, xt_b = x_target[:, :HW], x_target[:, HW:]
    xs_pad = jnp.pad(x_source, ((0, NPAD - N_SRC), (0, 0)))
    xt_pad = jnp.pad(x_target, ((0, NPAD - N_TGT), (0, 0)))

    sum_st_a, cnt_st, sum_ts_a, cnt_ts = _sc_pass1_counts()(
        xs_a, xt_a, st_e, ts_e)
    sum_st_b, sum_ts_b = _sc_pass1_nocounts()(
        xs_b, xt_b, st_e, ts_e)

    xt0a, xt0b, xs0 = pl.pallas_call(
        _tc_layer0_body,
        grid=(NPAD // _BLK,),
        in_specs=[
            _row_blk((HW,)), _row_blk((HW,)), _row_blk(()), _row_blk((D,)),
            _full((H, D)), _full((H,)), _full((H, D)),
            _row_blk((HW,)), _row_blk((HW,)), _row_blk(()), _row_blk((D,)),
            _full((H, D)), _full((H,)), _full((H, D)),
        ],
        out_specs=[_row_blk((HW,)), _row_blk((HW,)), _row_blk((H,))],
        out_shape=[jax.ShapeDtypeStruct((NPAD, HW), _f32),
                   jax.ShapeDtypeStruct((NPAD, HW), _f32),
                   jax.ShapeDtypeStruct((NPAD, H), _f32)],
    )(sum_st_a, sum_st_b, cnt_st, xt_pad, l0_st_Wl, l0_st_bl, l0_st_Wr,
      sum_ts_a, sum_ts_b, cnt_ts, xs_pad, l0_ts_Wl, l0_ts_bl, l0_ts_Wr)

    parts_a = _sc_pass3()(xt0a, ts3_e)
    parts_b = _sc_pass3()(xt0b, ts3_e)

    out = pl.pallas_call(
        _tc_layer1_body,
        grid=(NPAD // _BLK,),
        in_specs=[
            _row_blk((HW,)), _row_blk((HW,)), _row_blk((HW,)), _row_blk((HW,)),
            _row_blk(()), _row_blk((H,)),
            _full((H, H)), _full((H,)), _full((H, H)),
            _full((OUT, H)), _full((OUT,)),
        ],
        out_specs=_row_blk((OUT,)),
        out_shape=jax.ShapeDtypeStruct((NPAD, OUT), _f32),
    )(parts_a[0], parts_a[1], parts_b[0], parts_b[1], cnt_ts, xs0,
      l1_ts_Wl, l1_ts_bl, l1_ts_Wr, W_out, b_out)

    return out[:N_SRC]


# speedup vs baseline: 1.3111x; 1.3111x over previous
import functools

import jax
import jax.numpy as jnp
from jax import lax
from jax.experimental import pallas as pl
from jax.experimental.pallas import tpu as pltpu
from jax.experimental.pallas import tpu_sc as plsc

N_SRC = 10000
N_TGT = 10000
E = 320000
D = 128
H = 128
OUT = 64
HW = 64

NPAD = 10240
DUMMY = 10000
STRIPE = NPAD // 16
CH1 = 157

_f32 = jnp.float32
_i32 = jnp.int32


@functools.cache
def _mesh():
    return plsc.VectorSubcoreMesh(core_axis_name="c", subcore_axis_name="s")


def _zero_vec_loop(ref, n16):
    def body(i, _):
        ref[pl.ds(i * 16, 16)] = jnp.zeros((16,), _f32)
        return 0
    lax.fori_loop(0, n16, body, 0)


def _zero_rows(rows_v):
    n = 128 * HW // 16

    def body(i, _):
        r = i // (HW // 16)
        k = i % (HW // 16)
        rows_v[r, pl.ds(k * 16, 16)] = jnp.zeros((16,), _f32)
        return 0
    lax.fori_loop(0, n, body, 0)


def _fill_ones(ones_v):
    def body(i, _):
        ones_v[pl.ds(i * 16, 16)] = jnp.ones((16,), _f32)
        return 0
    lax.fori_loop(0, 8, body, 0)


def _accumulate(x_hbm, si_hbm, di_hbm, tile, nch, si_v, di_v, rows_a, rows_b,
                ones_v, acc_sh, cnt_sh, sem_a, sem_b):
    pltpu.sync_copy(si_hbm.at[tile], si_v)
    pltpu.sync_copy(di_hbm.at[tile], di_v)

    def start(j, buf, sem):
        pltpu.async_copy(x_hbm.at[si_v.at[j]], buf, sem)

    def wait(j, buf, sem):
        pltpu.make_async_copy(x_hbm.at[si_v.at[j]], buf, sem).wait()

    def scatter(j, buf):
        pltpu.sync_copy(buf, acc_sh.at[di_v.at[j]], add=True)
        if cnt_sh is not None:
            pltpu.sync_copy(ones_v, cnt_sh.at[di_v.at[j]], add=True)

    start(0, rows_a, sem_a)

    @pl.when(1 < nch)
    def _():
        start(1, rows_b, sem_b)

    def body(p, _):
        j = 2 * p
        wait(j, rows_a, sem_a)
        scatter(j, rows_a)

        @pl.when(j + 2 < nch)
        def _():
            start(j + 2, rows_a, sem_a)

        @pl.when(j + 1 < nch)
        def _():
            wait(j + 1, rows_b, sem_b)
            scatter(j + 1, rows_b)

        @pl.when(j + 3 < nch)
        def _():
            start(j + 3, rows_b, sem_b)
        return 0
    lax.fori_loop(0, (nch + 1) // 2, body, 0)


def _zero_acc_stripe(rows_v, acc_sh, t):
    for b in range(STRIPE // 128):
        pltpu.sync_copy(rows_v, acc_sh.at[pl.ds(t * STRIPE + b * 128, 128)])


def _p1_body_counts(xs_hbm, xt_hbm, st_si, st_di, ts_si, ts_di,
                    sum_st, cnt_st, sum_ts, cnt_ts,
                    si_v, di_v, rows_v, rows_w, ones_v, zc_v, acc_sh, cnt_sh,
                    sem_a, sem_b):
    c = lax.axis_index("c")
    t = lax.axis_index("s")

    _zero_rows(rows_v)
    _fill_ones(ones_v)
    _zero_vec_loop(zc_v, STRIPE // 16)
    _zero_acc_stripe(rows_v, acc_sh, t)
    pltpu.sync_copy(zc_v, cnt_sh.at[pl.ds(t * STRIPE, STRIPE)])
    plsc.subcore_barrier()

    @pl.when(c == 0)
    def _():
        _accumulate(xs_hbm, st_si, st_di, t, CH1, si_v, di_v, rows_v, rows_w,
                    ones_v, acc_sh, cnt_sh, sem_a, sem_b)

    @pl.when(c == 1)
    def _():
        _accumulate(xt_hbm, ts_si, ts_di, t, CH1, si_v, di_v, rows_v, rows_w,
                    ones_v, acc_sh, cnt_sh, sem_a, sem_b)

    plsc.subcore_barrier()

    @pl.when(c == 0)
    def _():
        pltpu.sync_copy(acc_sh.at[pl.ds(t * STRIPE, STRIPE)],
                        sum_st.at[pl.ds(t * STRIPE, STRIPE)])
        pltpu.sync_copy(cnt_sh.at[pl.ds(t * STRIPE, STRIPE)],
                        cnt_st.at[pl.ds(t * STRIPE, STRIPE)])

    @pl.when(c == 1)
    def _():
        pltpu.sync_copy(acc_sh.at[pl.ds(t * STRIPE, STRIPE)],
                        sum_ts.at[pl.ds(t * STRIPE, STRIPE)])
        pltpu.sync_copy(cnt_sh.at[pl.ds(t * STRIPE, STRIPE)],
                        cnt_ts.at[pl.ds(t * STRIPE, STRIPE)])


def _p1_body_nocounts(xs_hbm, xt_hbm, st_si, st_di, ts_si, ts_di,
                      sum_st, sum_ts,
                      si_v, di_v, rows_v, rows_w, acc_sh, sem_a, sem_b):
    c = lax.axis_index("c")
    t = lax.axis_index("s")

    _zero_rows(rows_v)
    _zero_acc_stripe(rows_v, acc_sh, t)
    plsc.subcore_barrier()

    @pl.when(c == 0)
    def _():
        _accumulate(xs_hbm, st_si, st_di, t, CH1, si_v, di_v, rows_v, rows_w,
                    None, acc_sh, None, sem_a, sem_b)

    @pl.when(c == 1)
    def _():
        _accumulate(xt_hbm, ts_si, ts_di, t, CH1, si_v, di_v, rows_v, rows_w,
                    None, acc_sh, None, sem_a, sem_b)

    plsc.subcore_barrier()

    @pl.when(c == 0)
    def _():
        pltpu.sync_copy(acc_sh.at[pl.ds(t * STRIPE, STRIPE)],
                        sum_st.at[pl.ds(t * STRIPE, STRIPE)])

    @pl.when(c == 1)
    def _():
        pltpu.sync_copy(acc_sh.at[pl.ds(t * STRIPE, STRIPE)],
                        sum_ts.at[pl.ds(t * STRIPE, STRIPE)])


@functools.cache
def _sc_pass1_counts():
    return pl.kernel(
        _p1_body_counts,
        out_type=(
            jax.ShapeDtypeStruct((NPAD, HW), _f32),
            jax.ShapeDtypeStruct((NPAD,), _f32),
            jax.ShapeDtypeStruct((NPAD, HW), _f32),
            jax.ShapeDtypeStruct((NPAD,), _f32),
        ),
        mesh=_mesh(),
        compiler_params=pltpu.CompilerParams(use_tc_tiling_on_sc=False),
        scratch_types=[
            pltpu.VMEM((CH1, 128), _i32),
            pltpu.VMEM((CH1, 128), _i32),
            pltpu.VMEM((128, HW), _f32),
            pltpu.VMEM((128, HW), _f32),
            pltpu.VMEM((128,), _f32),
            pltpu.VMEM((STRIPE,), _f32),
            pltpu.VMEM_SHARED((NPAD, HW), _f32),
            pltpu.VMEM_SHARED((NPAD,), _f32),
            pltpu.SemaphoreType.DMA,
            pltpu.SemaphoreType.DMA,
        ],
    )


@functools.cache
def _sc_pass1_nocounts():
    return pl.kernel(
        _p1_body_nocounts,
        out_type=(
            jax.ShapeDtypeStruct((NPAD, HW), _f32),
            jax.ShapeDtypeStruct((NPAD, HW), _f32),
        ),
        mesh=_mesh(),
        compiler_params=pltpu.CompilerParams(use_tc_tiling_on_sc=False),
        scratch_types=[
            pltpu.VMEM((CH1, 128), _i32),
            pltpu.VMEM((CH1, 128), _i32),
            pltpu.VMEM((128, HW), _f32),
            pltpu.VMEM((128, HW), _f32),
            pltpu.VMEM_SHARED((NPAD, HW), _f32),
            pltpu.SemaphoreType.DMA,
            pltpu.SemaphoreType.DMA,
        ],
    )


def _lrelu(x):
    return jnp.where(x >= 0, x, 0.01 * x)


_BLK = 2048


def _tc_layer0_body(sum_ta, sum_tb, cnt_t, x_t, wl_t, bl_t, wr_t,
                    sum_sa, sum_sb, cnt_s, x_s, wl_s, bl_s, wr_s,
                    xt0a, xt0b, xs0):
    rt = 1.0 / jnp.maximum(cnt_t[...], 1.0)
    mean_t = jnp.concatenate([sum_ta[...], sum_tb[...]], axis=1) * rt[:, None]
    h_t = (jnp.dot(mean_t, wl_t[...].T, preferred_element_type=_f32)
           + bl_t[...][None, :]
           + jnp.dot(x_t[...], wr_t[...].T, preferred_element_type=_f32))
    a_t = _lrelu(h_t)
    xt0a[...] = a_t[:, :HW]
    xt0b[...] = a_t[:, HW:]
    rs = 1.0 / jnp.maximum(cnt_s[...], 1.0)
    mean_s = jnp.concatenate([sum_sa[...], sum_sb[...]], axis=1) * rs[:, None]
    h_s = (jnp.dot(mean_s, wl_s[...].T, preferred_element_type=_f32)
           + bl_s[...][None, :]
           + jnp.dot(x_s[...], wr_s[...].T, preferred_element_type=_f32))
    xs0[...] = _lrelu(h_s)


def _tc_layer1_body(sa, sb, cnt, xs0, wl, bl, wr, wo, bo, out):
    r = 1.0 / jnp.maximum(cnt[...], 1.0)
    mean = jnp.concatenate([sa[...], sb[...]], axis=1) * r[:, None]
    h = (jnp.dot(mean, wl[...].T, preferred_element_type=_f32)
         + bl[...][None, :]
         + jnp.dot(xs0[...], wr[...].T, preferred_element_type=_f32))
    xs1 = _lrelu(h)
    out[...] = (jnp.dot(xs1, wo[...].T, preferred_element_type=_f32)
                + bo[...][None, :])


def _row_blk(shape_minor):
    return pl.BlockSpec((_BLK,) + shape_minor,
                        lambda i: (i,) + (0,) * len(shape_minor))


def _full(shape):
    return pl.BlockSpec(shape, lambda i: (0,) * len(shape))


def _prep_edges(ei, ntiles, nch):
    epad = ntiles * nch * 128
    src = jnp.zeros((epad,), _i32).at[:E].set(ei[0].astype(_i32))
    dst = jnp.full((epad,), DUMMY, _i32).at[:E].set(ei[1].astype(_i32))
    return src.reshape(ntiles, nch, 128), dst.reshape(ntiles, nch, 128)


def kernel(x_source, x_target, edge_index_st, edge_index_ts,
           l0_st_Wl, l0_st_bl, l0_st_Wr, l0_ts_Wl, l0_ts_bl, l0_ts_Wr,
           l1_st_Wl, l1_st_bl, l1_st_Wr, l1_ts_Wl, l1_ts_bl, l1_ts_Wr,
           W_out, b_out):
    st_si, st_di = _prep_edges(edge_index_st, 16, CH1)
    ts_si, ts_di = _prep_edges(edge_index_ts, 16, CH1)

    xs_a, xs_b = x_source[:, :HW], x_source[:, HW:]
    xt_a, xt_b = x_target[:, :HW], x_target[:, HW:]
    xs_pad = jnp.pad(x_source, ((0, NPAD - N_SRC), (0, 0)))
    xt_pad = jnp.pad(x_target, ((0, NPAD - N_TGT), (0, 0)))

    sum_st_a, cnt_st, sum_ts_a, cnt_ts = _sc_pass1_counts()(
        xs_a, xt_a, st_si, st_di, ts_si, ts_di)
    sum_st_b, sum_ts_b = _sc_pass1_nocounts()(
        xs_b, xt_b, st_si, st_di, ts_si, ts_di)

    xt0a, xt0b, xs0 = pl.pallas_call(
        _tc_layer0_body,
        grid=(NPAD // _BLK,),
        in_specs=[
            _row_blk((HW,)), _row_blk((HW,)), _row_blk(()), _row_blk((D,)),
            _full((H, D)), _full((H,)), _full((H, D)),
            _row_blk((HW,)), _row_blk((HW,)), _row_blk(()), _row_blk((D,)),
            _full((H, D)), _full((H,)), _full((H, D)),
        ],
        out_specs=[_row_blk((HW,)), _row_blk((HW,)), _row_blk((H,))],
        out_shape=[jax.ShapeDtypeStruct((NPAD, HW), _f32),
                   jax.ShapeDtypeStruct((NPAD, HW), _f32),
                   jax.ShapeDtypeStruct((NPAD, H), _f32)],
    )(sum_st_a, sum_st_b, cnt_st, xt_pad, l0_st_Wl, l0_st_bl, l0_st_Wr,
      sum_ts_a, sum_ts_b, cnt_ts, xs_pad, l0_ts_Wl, l0_ts_bl, l0_ts_Wr)

    sum_l1_a, sum_l1_b = _sc_pass1_nocounts()(
        xt0a, xt0b, ts_si, ts_di, ts_si, ts_di)

    out = pl.pallas_call(
        _tc_layer1_body,
        grid=(NPAD // _BLK,),
        in_specs=[
            _row_blk((HW,)), _row_blk((HW,)),
            _row_blk(()), _row_blk((H,)),
            _full((H, H)), _full((H,)), _full((H, H)),
            _full((OUT, H)), _full((OUT,)),
        ],
        out_specs=_row_blk((OUT,)),
        out_shape=jax.ShapeDtypeStruct((NPAD, OUT), _f32),
    )(sum_l1_a, sum_l1_b, cnt_ts, xs0,
      l1_ts_Wl, l1_ts_bl, l1_ts_Wr, W_out, b_out)

    return out[:N_SRC]


# speedup vs baseline: 1.3756x; 1.0492x over previous
import functools

import jax
import jax.numpy as jnp
from jax import lax
from jax.experimental import pallas as pl
from jax.experimental.pallas import tpu as pltpu
from jax.experimental.pallas import tpu_sc as plsc

N_SRC = 10000
N_TGT = 10000
E = 320000
D = 128
H = 128
OUT = 64
HW = 64

NPAD = 10240
DUMMY = 10000
STRIPE = NPAD // 16
CH1 = 157

_f32 = jnp.float32
_i32 = jnp.int32


@functools.cache
def _mesh():
    return plsc.VectorSubcoreMesh(core_axis_name="c", subcore_axis_name="s")


def _zero_vec_loop(ref, n16):
    def body(i, _):
        ref[pl.ds(i * 16, 16)] = jnp.zeros((16,), _f32)
        return 0
    lax.fori_loop(0, n16, body, 0)


def _zero_rows(rows_v):
    n = 128 * HW // 16

    def body(i, _):
        r = i // (HW // 16)
        k = i % (HW // 16)
        rows_v[r, pl.ds(k * 16, 16)] = jnp.zeros((16,), _f32)
        return 0
    lax.fori_loop(0, n, body, 0)


def _fill_ones(ones_v):
    def body(i, _):
        ones_v[pl.ds(i * 16, 16)] = jnp.ones((16,), _f32)
        return 0
    lax.fori_loop(0, 8, body, 0)


def _accumulate(x_hbm, si_hbm, di_hbm, tile, nch, si_v, di_v, rows_a, rows_b,
                ones_v, acc_sh, cnt_sh, sem_a, sem_b):
    pltpu.sync_copy(si_hbm.at[tile], si_v)
    pltpu.sync_copy(di_hbm.at[tile], di_v)

    def start(j, buf, sem):
        pltpu.async_copy(x_hbm.at[si_v.at[j]], buf, sem)

    def wait(j, buf, sem):
        pltpu.make_async_copy(x_hbm.at[si_v.at[j]], buf, sem).wait()

    def scatter(j, buf):
        pltpu.sync_copy(buf, acc_sh.at[di_v.at[j]], add=True)
        if cnt_sh is not None:
            pltpu.sync_copy(ones_v, cnt_sh.at[di_v.at[j]], add=True)

    start(0, rows_a, sem_a)

    @pl.when(1 < nch)
    def _():
        start(1, rows_b, sem_b)

    def body(p, _):
        j = 2 * p
        wait(j, rows_a, sem_a)
        scatter(j, rows_a)

        @pl.when(j + 2 < nch)
        def _():
            start(j + 2, rows_a, sem_a)

        @pl.when(j + 1 < nch)
        def _():
            wait(j + 1, rows_b, sem_b)
            scatter(j + 1, rows_b)

        @pl.when(j + 3 < nch)
        def _():
            start(j + 3, rows_b, sem_b)
        return 0
    lax.fori_loop(0, (nch + 1) // 2, body, 0)


def _zero_acc_stripe(rows_v, acc_sh, t):
    for b in range(STRIPE // 128):
        pltpu.sync_copy(rows_v, acc_sh.at[pl.ds(t * STRIPE + b * 128, 128)])


def _conv_body_counts(xa_hbm, xb_hbm, e_si, e_di,
                      sum_a, sum_b, cnt,
                      si_v, di_v, rows_v, rows_w, ones_v, zc_v, acc_sh,
                      cnt_sh, sem_a, sem_b):
    c = lax.axis_index("c")
    t = lax.axis_index("s")

    _zero_rows(rows_v)
    _fill_ones(ones_v)
    _zero_vec_loop(zc_v, STRIPE // 16)
    _zero_acc_stripe(rows_v, acc_sh, t)

    @pl.when(c == 0)
    def _():
        pltpu.sync_copy(zc_v, cnt_sh.at[pl.ds(t * STRIPE, STRIPE)])

    plsc.subcore_barrier()

    @pl.when(c == 0)
    def _():
        _accumulate(xa_hbm, e_si, e_di, t, CH1, si_v, di_v, rows_v, rows_w,
                    ones_v, acc_sh, cnt_sh, sem_a, sem_b)

    @pl.when(c == 1)
    def _():
        _accumulate(xb_hbm, e_si, e_di, t, CH1, si_v, di_v, rows_v, rows_w,
                    None, acc_sh, None, sem_a, sem_b)

    plsc.subcore_barrier()

    @pl.when(c == 0)
    def _():
        pltpu.sync_copy(acc_sh.at[pl.ds(t * STRIPE, STRIPE)],
                        sum_a.at[pl.ds(t * STRIPE, STRIPE)])
        pltpu.sync_copy(cnt_sh.at[pl.ds(t * STRIPE, STRIPE)],
                        cnt.at[pl.ds(t * STRIPE, STRIPE)])

    @pl.when(c == 1)
    def _():
        pltpu.sync_copy(acc_sh.at[pl.ds(t * STRIPE, STRIPE)],
                        sum_b.at[pl.ds(t * STRIPE, STRIPE)])


def _p1_body_nocounts(xs_hbm, xt_hbm, st_si, st_di, ts_si, ts_di,
                      sum_st, sum_ts,
                      si_v, di_v, rows_v, rows_w, acc_sh, sem_a, sem_b):
    c = lax.axis_index("c")
    t = lax.axis_index("s")

    _zero_rows(rows_v)
    _zero_acc_stripe(rows_v, acc_sh, t)
    plsc.subcore_barrier()

    @pl.when(c == 0)
    def _():
        _accumulate(xs_hbm, st_si, st_di, t, CH1, si_v, di_v, rows_v, rows_w,
                    None, acc_sh, None, sem_a, sem_b)

    @pl.when(c == 1)
    def _():
        _accumulate(xt_hbm, ts_si, ts_di, t, CH1, si_v, di_v, rows_v, rows_w,
                    None, acc_sh, None, sem_a, sem_b)

    plsc.subcore_barrier()

    @pl.when(c == 0)
    def _():
        pltpu.sync_copy(acc_sh.at[pl.ds(t * STRIPE, STRIPE)],
                        sum_st.at[pl.ds(t * STRIPE, STRIPE)])

    @pl.when(c == 1)
    def _():
        pltpu.sync_copy(acc_sh.at[pl.ds(t * STRIPE, STRIPE)],
                        sum_ts.at[pl.ds(t * STRIPE, STRIPE)])


@functools.cache
def _sc_conv_counts():
    return pl.kernel(
        _conv_body_counts,
        out_type=(
            jax.ShapeDtypeStruct((NPAD, HW), _f32),
            jax.ShapeDtypeStruct((NPAD, HW), _f32),
            jax.ShapeDtypeStruct((NPAD,), _f32),
        ),
        mesh=_mesh(),
        compiler_params=pltpu.CompilerParams(use_tc_tiling_on_sc=False),
        scratch_types=[
            pltpu.VMEM((CH1, 128), _i32),
            pltpu.VMEM((CH1, 128), _i32),
            pltpu.VMEM((128, HW), _f32),
            pltpu.VMEM((128, HW), _f32),
            pltpu.VMEM((128,), _f32),
            pltpu.VMEM((STRIPE,), _f32),
            pltpu.VMEM_SHARED((NPAD, HW), _f32),
            pltpu.VMEM_SHARED((NPAD,), _f32),
            pltpu.SemaphoreType.DMA,
            pltpu.SemaphoreType.DMA,
        ],
    )


@functools.cache
def _sc_pass1_nocounts():
    return pl.kernel(
        _p1_body_nocounts,
        out_type=(
            jax.ShapeDtypeStruct((NPAD, HW), _f32),
            jax.ShapeDtypeStruct((NPAD, HW), _f32),
        ),
        mesh=_mesh(),
        compiler_params=pltpu.CompilerParams(use_tc_tiling_on_sc=False),
        scratch_types=[
            pltpu.VMEM((CH1, 128), _i32),
            pltpu.VMEM((CH1, 128), _i32),
            pltpu.VMEM((128, HW), _f32),
            pltpu.VMEM((128, HW), _f32),
            pltpu.VMEM_SHARED((NPAD, HW), _f32),
            pltpu.SemaphoreType.DMA,
            pltpu.SemaphoreType.DMA,
        ],
    )


def _lrelu(x):
    return jnp.where(x >= 0, x, 0.01 * x)


_BLK = 2048


def _tc_conv_split_body(sum_a, sum_b, cnt, x, wl, bl, wr, outa, outb):
    r = 1.0 / jnp.maximum(cnt[...], 1.0)
    mean = jnp.concatenate([sum_a[...], sum_b[...]], axis=1) * r[:, None]
    h = (jnp.dot(mean, wl[...].T, preferred_element_type=_f32)
         + bl[...][None, :]
         + jnp.dot(x[...], wr[...].T, preferred_element_type=_f32))
    a = _lrelu(h)
    outa[...] = a[:, :HW]
    outb[...] = a[:, HW:]


def _tc_conv_body(sum_a, sum_b, cnt, x, wl, bl, wr, out):
    r = 1.0 / jnp.maximum(cnt[...], 1.0)
    mean = jnp.concatenate([sum_a[...], sum_b[...]], axis=1) * r[:, None]
    h = (jnp.dot(mean, wl[...].T, preferred_element_type=_f32)
         + bl[...][None, :]
         + jnp.dot(x[...], wr[...].T, preferred_element_type=_f32))
    out[...] = _lrelu(h)


def _tc_layer1_body(sa, sb, cnt, xs0, wl, bl, wr, wo, bo, out):
    r = 1.0 / jnp.maximum(cnt[...], 1.0)
    mean = jnp.concatenate([sa[...], sb[...]], axis=1) * r[:, None]
    h = (jnp.dot(mean, wl[...].T, preferred_element_type=_f32)
         + bl[...][None, :]
         + jnp.dot(xs0[...], wr[...].T, preferred_element_type=_f32))
    xs1 = _lrelu(h)
    out[...] = (jnp.dot(xs1, wo[...].T, preferred_element_type=_f32)
                + bo[...][None, :])


def _row_blk(shape_minor):
    return pl.BlockSpec((_BLK,) + shape_minor,
                        lambda i: (i,) + (0,) * len(shape_minor))


def _full(shape):
    return pl.BlockSpec(shape, lambda i: (0,) * len(shape))


def _prep_edges(ei, ntiles, nch):
    epad = ntiles * nch * 128
    src = jnp.zeros((epad,), _i32).at[:E].set(ei[0].astype(_i32))
    dst = jnp.full((epad,), DUMMY, _i32).at[:E].set(ei[1].astype(_i32))
    return src.reshape(ntiles, nch, 128), dst.reshape(ntiles, nch, 128)


def kernel(x_source, x_target, edge_index_st, edge_index_ts,
           l0_st_Wl, l0_st_bl, l0_st_Wr, l0_ts_Wl, l0_ts_bl, l0_ts_Wr,
           l1_st_Wl, l1_st_bl, l1_st_Wr, l1_ts_Wl, l1_ts_bl, l1_ts_Wr,
           W_out, b_out):
    st_si, st_di = _prep_edges(edge_index_st, 16, CH1)
    ts_si, ts_di = _prep_edges(edge_index_ts, 16, CH1)

    xs_a, xs_b = x_source[:, :HW], x_source[:, HW:]
    xt_a, xt_b = x_target[:, :HW], x_target[:, HW:]
    xs_pad = jnp.pad(x_source, ((0, NPAD - N_SRC), (0, 0)))
    xt_pad = jnp.pad(x_target, ((0, NPAD - N_TGT), (0, 0)))

    sum_st_a, sum_st_b, cnt_st = _sc_conv_counts()(
        xs_a, xs_b, st_si, st_di)
    sum_ts_a, sum_ts_b, cnt_ts = _sc_conv_counts()(
        xt_a, xt_b, ts_si, ts_di)

    conv_specs = [
        _row_blk((HW,)), _row_blk((HW,)), _row_blk(()), _row_blk((D,)),
        _full((H, D)), _full((H,)), _full((H, D)),
    ]
    xt0a, xt0b = pl.pallas_call(
        _tc_conv_split_body,
        grid=(NPAD // _BLK,),
        in_specs=conv_specs,
        out_specs=[_row_blk((HW,)), _row_blk((HW,))],
        out_shape=[jax.ShapeDtypeStruct((NPAD, HW), _f32),
                   jax.ShapeDtypeStruct((NPAD, HW), _f32)],
    )(sum_st_a, sum_st_b, cnt_st, xt_pad, l0_st_Wl, l0_st_bl, l0_st_Wr)

    xs0 = pl.pallas_call(
        _tc_conv_body,
        grid=(NPAD // _BLK,),
        in_specs=conv_specs,
        out_specs=_row_blk((H,)),
        out_shape=jax.ShapeDtypeStruct((NPAD, H), _f32),
    )(sum_ts_a, sum_ts_b, cnt_ts, xs_pad, l0_ts_Wl, l0_ts_bl, l0_ts_Wr)

    sum_l1_a, sum_l1_b = _sc_pass1_nocounts()(
        xt0a, xt0b, ts_si, ts_di, ts_si, ts_di)

    out = pl.pallas_call(
        _tc_layer1_body,
        grid=(NPAD // _BLK,),
        in_specs=[
            _row_blk((HW,)), _row_blk((HW,)),
            _row_blk(()), _row_blk((H,)),
            _full((H, H)), _full((H,)), _full((H, H)),
            _full((OUT, H)), _full((OUT,)),
        ],
        out_specs=_row_blk((OUT,)),
        out_shape=jax.ShapeDtypeStruct((NPAD, OUT), _f32),
    )(sum_l1_a, sum_l1_b, cnt_ts, xs0,
      l1_ts_Wl, l1_ts_bl, l1_ts_Wr, W_out, b_out)

    return out[:N_SRC]


# speedup vs baseline: 1.4299x; 1.0394x over previous
import functools

import jax
import jax.numpy as jnp
from jax import lax
from jax.experimental import pallas as pl
from jax.experimental.pallas import tpu as pltpu
from jax.experimental.pallas import tpu_sc as plsc

N_SRC = 10000
N_TGT = 10000
E = 320000
D = 128
H = 128
OUT = 64
HW = 64

NPAD = 10240
DUMMY = 10000
STRIPE = NPAD // 16
CH1 = 157

_f32 = jnp.float32
_i32 = jnp.int32


@functools.cache
def _mesh():
    return plsc.VectorSubcoreMesh(core_axis_name="c", subcore_axis_name="s")


def _zero_vec_loop(ref, n16):
    def body(i, _):
        ref[pl.ds(i * 16, 16)] = jnp.zeros((16,), _f32)
        return 0
    lax.fori_loop(0, n16, body, 0)


def _zero_rows(rows_v):
    n = 128 * HW // 16

    def body(i, _):
        r = i // (HW // 16)
        k = i % (HW // 16)
        rows_v[r, pl.ds(k * 16, 16)] = jnp.zeros((16,), _f32)
        return 0
    lax.fori_loop(0, n, body, 0)


def _fill_ones(ones_v):
    def body(i, _):
        ones_v[pl.ds(i * 16, 16)] = jnp.ones((16,), _f32)
        return 0
    lax.fori_loop(0, 8, body, 0)


def _accumulate(x_hbm, si_hbm, di_hbm, tile, nch, si_v, di_v, rows_a, rows_b,
                ones_v, acc_sh, cnt_sh, sem_a, sem_b, clo=None, chi=None):
    pltpu.sync_copy(si_hbm.at[tile], si_v)
    pltpu.sync_copy(di_hbm.at[tile], di_v)

    def start(j, buf, sem):
        pltpu.async_copy(x_hbm.at[si_v.at[j]], buf, sem)

    def wait(j, buf, sem):
        pltpu.make_async_copy(x_hbm.at[si_v.at[j]], buf, sem).wait()

    def scatter(j, buf):
        pltpu.sync_copy(buf, acc_sh.at[di_v.at[j]], add=True)
        if cnt_sh is not None:
            @pl.when((j >= clo) & (j < chi))
            def _():
                pltpu.sync_copy(ones_v, cnt_sh.at[di_v.at[j]], add=True)

    start(0, rows_a, sem_a)

    @pl.when(1 < nch)
    def _():
        start(1, rows_b, sem_b)

    def body(p, _):
        j = 2 * p
        wait(j, rows_a, sem_a)
        scatter(j, rows_a)

        @pl.when(j + 2 < nch)
        def _():
            start(j + 2, rows_a, sem_a)

        @pl.when(j + 1 < nch)
        def _():
            wait(j + 1, rows_b, sem_b)
            scatter(j + 1, rows_b)

        @pl.when(j + 3 < nch)
        def _():
            start(j + 3, rows_b, sem_b)
        return 0
    lax.fori_loop(0, (nch + 1) // 2, body, 0)


def _zero_acc_stripe(rows_v, acc_sh, t):
    for b in range(STRIPE // 128):
        pltpu.sync_copy(rows_v, acc_sh.at[pl.ds(t * STRIPE + b * 128, 128)])


CHALF = CH1 // 2 + 1


def _conv_body_counts(xa_hbm, xb_hbm, e_si, e_di,
                      sum_a, sum_b, cnt_a, cnt_b,
                      si_v, di_v, rows_v, rows_w, ones_v, zc_v, acc_sh,
                      cnt_sh, sem_a, sem_b):
    c = lax.axis_index("c")
    t = lax.axis_index("s")

    _zero_rows(rows_v)
    _fill_ones(ones_v)
    _zero_vec_loop(zc_v, STRIPE // 16)
    _zero_acc_stripe(rows_v, acc_sh, t)
    pltpu.sync_copy(zc_v, cnt_sh.at[pl.ds(t * STRIPE, STRIPE)])
    plsc.subcore_barrier()

    clo = jnp.where(c == 0, 0, CHALF)
    chi = jnp.where(c == 0, CHALF, CH1)

    @pl.when(c == 0)
    def _():
        _accumulate(xa_hbm, e_si, e_di, t, CH1, si_v, di_v, rows_v, rows_w,
                    ones_v, acc_sh, cnt_sh, sem_a, sem_b, clo, chi)

    @pl.when(c == 1)
    def _():
        _accumulate(xb_hbm, e_si, e_di, t, CH1, si_v, di_v, rows_v, rows_w,
                    ones_v, acc_sh, cnt_sh, sem_a, sem_b, clo, chi)

    plsc.subcore_barrier()

    @pl.when(c == 0)
    def _():
        pltpu.sync_copy(acc_sh.at[pl.ds(t * STRIPE, STRIPE)],
                        sum_a.at[pl.ds(t * STRIPE, STRIPE)])
        pltpu.sync_copy(cnt_sh.at[pl.ds(t * STRIPE, STRIPE)],
                        cnt_a.at[pl.ds(t * STRIPE, STRIPE)])

    @pl.when(c == 1)
    def _():
        pltpu.sync_copy(acc_sh.at[pl.ds(t * STRIPE, STRIPE)],
                        sum_b.at[pl.ds(t * STRIPE, STRIPE)])
        pltpu.sync_copy(cnt_sh.at[pl.ds(t * STRIPE, STRIPE)],
                        cnt_b.at[pl.ds(t * STRIPE, STRIPE)])


def _p1_body_nocounts(xs_hbm, xt_hbm, st_si, st_di, ts_si, ts_di,
                      sum_st, sum_ts,
                      si_v, di_v, rows_v, rows_w, acc_sh, sem_a, sem_b):
    c = lax.axis_index("c")
    t = lax.axis_index("s")

    _zero_rows(rows_v)
    _zero_acc_stripe(rows_v, acc_sh, t)
    plsc.subcore_barrier()

    @pl.when(c == 0)
    def _():
        _accumulate(xs_hbm, st_si, st_di, t, CH1, si_v, di_v, rows_v, rows_w,
                    None, acc_sh, None, sem_a, sem_b)

    @pl.when(c == 1)
    def _():
        _accumulate(xt_hbm, ts_si, ts_di, t, CH1, si_v, di_v, rows_v, rows_w,
                    None, acc_sh, None, sem_a, sem_b)

    plsc.subcore_barrier()

    @pl.when(c == 0)
    def _():
        pltpu.sync_copy(acc_sh.at[pl.ds(t * STRIPE, STRIPE)],
                        sum_st.at[pl.ds(t * STRIPE, STRIPE)])

    @pl.when(c == 1)
    def _():
        pltpu.sync_copy(acc_sh.at[pl.ds(t * STRIPE, STRIPE)],
                        sum_ts.at[pl.ds(t * STRIPE, STRIPE)])


@functools.cache
def _sc_conv_counts():
    return pl.kernel(
        _conv_body_counts,
        out_type=(
            jax.ShapeDtypeStruct((NPAD, HW), _f32),
            jax.ShapeDtypeStruct((NPAD, HW), _f32),
            jax.ShapeDtypeStruct((NPAD,), _f32),
            jax.ShapeDtypeStruct((NPAD,), _f32),
        ),
        mesh=_mesh(),
        compiler_params=pltpu.CompilerParams(use_tc_tiling_on_sc=False),
        scratch_types=[
            pltpu.VMEM((CH1, 128), _i32),
            pltpu.VMEM((CH1, 128), _i32),
            pltpu.VMEM((128, HW), _f32),
            pltpu.VMEM((128, HW), _f32),
            pltpu.VMEM((128,), _f32),
            pltpu.VMEM((STRIPE,), _f32),
            pltpu.VMEM_SHARED((NPAD, HW), _f32),
            pltpu.VMEM_SHARED((NPAD,), _f32),
            pltpu.SemaphoreType.DMA,
            pltpu.SemaphoreType.DMA,
        ],
    )


@functools.cache
def _sc_pass1_nocounts():
    return pl.kernel(
        _p1_body_nocounts,
        out_type=(
            jax.ShapeDtypeStruct((NPAD, HW), _f32),
            jax.ShapeDtypeStruct((NPAD, HW), _f32),
        ),
        mesh=_mesh(),
        compiler_params=pltpu.CompilerParams(use_tc_tiling_on_sc=False),
        scratch_types=[
            pltpu.VMEM((CH1, 128), _i32),
            pltpu.VMEM((CH1, 128), _i32),
            pltpu.VMEM((128, HW), _f32),
            pltpu.VMEM((128, HW), _f32),
            pltpu.VMEM_SHARED((NPAD, HW), _f32),
            pltpu.SemaphoreType.DMA,
            pltpu.SemaphoreType.DMA,
        ],
    )


def _lrelu(x):
    return jnp.where(x >= 0, x, 0.01 * x)


_BLK = 2048


def _tc_conv_split_body(sum_a, sum_b, ca, cb, x, wl, bl, wr, outa, outb):
    r = 1.0 / jnp.maximum(ca[...] + cb[...], 1.0)
    mean = jnp.concatenate([sum_a[...], sum_b[...]], axis=1) * r[:, None]
    h = (jnp.dot(mean, wl[...].T, preferred_element_type=_f32)
         + bl[...][None, :]
         + jnp.dot(x[...], wr[...].T, preferred_element_type=_f32))
    a = _lrelu(h)
    outa[...] = a[:, :HW]
    outb[...] = a[:, HW:]


def _tc_conv_body(sum_a, sum_b, ca, cb, x, wl, bl, wr, out):
    r = 1.0 / jnp.maximum(ca[...] + cb[...], 1.0)
    mean = jnp.concatenate([sum_a[...], sum_b[...]], axis=1) * r[:, None]
    h = (jnp.dot(mean, wl[...].T, preferred_element_type=_f32)
         + bl[...][None, :]
         + jnp.dot(x[...], wr[...].T, preferred_element_type=_f32))
    out[...] = _lrelu(h)


def _tc_layer1_body(sa, sb, ca, cb, xs0, wl, bl, wr, wo, bo, out):
    r = 1.0 / jnp.maximum(ca[...] + cb[...], 1.0)
    mean = jnp.concatenate([sa[...], sb[...]], axis=1) * r[:, None]
    h = (jnp.dot(mean, wl[...].T, preferred_element_type=_f32)
         + bl[...][None, :]
         + jnp.dot(xs0[...], wr[...].T, preferred_element_type=_f32))
    xs1 = _lrelu(h)
    out[...] = (jnp.dot(xs1, wo[...].T, preferred_element_type=_f32)
                + bo[...][None, :])


def _row_blk(shape_minor):
    return pl.BlockSpec((_BLK,) + shape_minor,
                        lambda i: (i,) + (0,) * len(shape_minor))


def _full(shape):
    return pl.BlockSpec(shape, lambda i: (0,) * len(shape))


def _prep_edges(ei, ntiles, nch):
    epad = ntiles * nch * 128
    src = jnp.zeros((epad,), _i32).at[:E].set(ei[0].astype(_i32))
    dst = jnp.full((epad,), DUMMY, _i32).at[:E].set(ei[1].astype(_i32))
    return src.reshape(ntiles, nch, 128), dst.reshape(ntiles, nch, 128)


def kernel(x_source, x_target, edge_index_st, edge_index_ts,
           l0_st_Wl, l0_st_bl, l0_st_Wr, l0_ts_Wl, l0_ts_bl, l0_ts_Wr,
           l1_st_Wl, l1_st_bl, l1_st_Wr, l1_ts_Wl, l1_ts_bl, l1_ts_Wr,
           W_out, b_out):
    st_si, st_di = _prep_edges(edge_index_st, 16, CH1)
    ts_si, ts_di = _prep_edges(edge_index_ts, 16, CH1)

    xs_a, xs_b = x_source[:, :HW], x_source[:, HW:]
    xt_a, xt_b = x_target[:, :HW], x_target[:, HW:]
    xs_pad = jnp.pad(x_source, ((0, NPAD - N_SRC), (0, 0)))
    xt_pad = jnp.pad(x_target, ((0, NPAD - N_TGT), (0, 0)))

    sum_st_a, sum_st_b, cst_a, cst_b = _sc_conv_counts()(
        xs_a, xs_b, st_si, st_di)
    sum_ts_a, sum_ts_b, cts_a, cts_b = _sc_conv_counts()(
        xt_a, xt_b, ts_si, ts_di)

    conv_specs = [
        _row_blk((HW,)), _row_blk((HW,)), _row_blk(()), _row_blk(()),
        _row_blk((D,)),
        _full((H, D)), _full((H,)), _full((H, D)),
    ]
    xt0a, xt0b = pl.pallas_call(
        _tc_conv_split_body,
        grid=(NPAD // _BLK,),
        in_specs=conv_specs,
        out_specs=[_row_blk((HW,)), _row_blk((HW,))],
        out_shape=[jax.ShapeDtypeStruct((NPAD, HW), _f32),
                   jax.ShapeDtypeStruct((NPAD, HW), _f32)],
    )(sum_st_a, sum_st_b, cst_a, cst_b, xt_pad, l0_st_Wl, l0_st_bl,
      l0_st_Wr)

    xs0 = pl.pallas_call(
        _tc_conv_body,
        grid=(NPAD // _BLK,),
        in_specs=conv_specs,
        out_specs=_row_blk((H,)),
        out_shape=jax.ShapeDtypeStruct((NPAD, H), _f32),
    )(sum_ts_a, sum_ts_b, cts_a, cts_b, xs_pad, l0_ts_Wl, l0_ts_bl,
      l0_ts_Wr)

    sum_l1_a, sum_l1_b = _sc_pass1_nocounts()(
        xt0a, xt0b, ts_si, ts_di, ts_si, ts_di)

    out = pl.pallas_call(
        _tc_layer1_body,
        grid=(NPAD // _BLK,),
        in_specs=[
            _row_blk((HW,)), _row_blk((HW,)),
            _row_blk(()), _row_blk(()), _row_blk((H,)),
            _full((H, H)), _full((H,)), _full((H, H)),
            _full((OUT, H)), _full((OUT,)),
        ],
        out_specs=_row_blk((OUT,)),
        out_shape=jax.ShapeDtypeStruct((NPAD, OUT), _f32),
    )(sum_l1_a, sum_l1_b, cts_a, cts_b, xs0,
      l1_ts_Wl, l1_ts_bl, l1_ts_Wr, W_out, b_out)

    return out[:N_SRC]
